# baseline (speedup 1.0000x reference)
import jax
import jax.numpy as jnp
from jax import lax
from jax.experimental import pallas as pl
from jax.experimental.pallas import tpu as pltpu
from jax.experimental.pallas import tpu_sc as plsc

V = 1000000
D = 64
B = 4096 * 200

_info = plsc.get_sparse_core_info()
NC = _info.num_cores
NS = _info.num_subcores
NW = NC * NS
IW = 128
B_PER_W = B // NW
CHUNK = 256
GPC = CHUNK // IW
CHUNKS = B_PER_W // CHUNK
IDX_ROWS_PER_W = B_PER_W // IW
NBUF = 4


def _sc_gather(table_hbm, idx_hbm, out_hbm, idx_v, rows_v,
               g0, g1, g2, g3, s0, s1, s2, s3):
    wid = lax.axis_index("s") * NC + lax.axis_index("c")
    idx_row_base = wid * IDX_ROWS_PER_W
    out_base = wid * B_PER_W
    gsems = (g0, g1, g2, g3)
    ssems = (s0, s1, s2, s3)

    pltpu.sync_copy(idx_hbm.at[pl.ds(idx_row_base, IDX_ROWS_PER_W)], idx_v)

    def fire_gathers(g, b):
        for j in range(GPC):
            pltpu.async_copy(
                table_hbm.at[idx_v.at[g * GPC + j]],
                rows_v.at[b].at[pl.ds(j * IW, IW)],
                gsems[b],
            )

    def drain_gathers(b):
        pltpu.make_async_copy(
            table_hbm.at[pl.ds(0, CHUNK)], rows_v.at[b], gsems[b]
        ).wait()

    def store_async(g, b):
        pltpu.async_copy(
            rows_v.at[b],
            out_hbm.at[pl.ds(out_base + g * CHUNK, CHUNK)],
            ssems[b],
        )

    def drain_store(b):
        pltpu.make_async_copy(
            rows_v.at[b], out_hbm.at[pl.ds(out_base, CHUNK)], ssems[b]
        ).wait()

    for b in range(NBUF - 1):
        fire_gathers(b, b)

    @pl.loop(0, CHUNKS, step=NBUF)
    def _quad(g):
        for k in range(NBUF):
            b = k
            drain_gathers(b)
            store_async(g + k, b)
            nb = (k + NBUF - 1) % NBUF
            nxt = g + k + NBUF - 1

            @pl.when(g + k > 0)
            def _():
                drain_store(nb)

            @pl.when(nxt < CHUNKS)
            def _():
                fire_gathers(nxt, nb)

    drain_store(NBUF - 1)


@jax.jit
def _lookup(indices_flat2d, table):
    mesh = plsc.VectorSubcoreMesh(core_axis_name="c", subcore_axis_name="s")
    run = pl.kernel(
        _sc_gather,
        out_type=jax.ShapeDtypeStruct((B, D), jnp.float32),
        mesh=mesh,
        scratch_types=[
            pltpu.VMEM((IDX_ROWS_PER_W, IW), jnp.int32),
            pltpu.VMEM((NBUF, CHUNK, D), jnp.float32),
            pltpu.SemaphoreType.DMA,
            pltpu.SemaphoreType.DMA,
            pltpu.SemaphoreType.DMA,
            pltpu.SemaphoreType.DMA,
            pltpu.SemaphoreType.DMA,
            pltpu.SemaphoreType.DMA,
            pltpu.SemaphoreType.DMA,
            pltpu.SemaphoreType.DMA,
        ],
        compiler_params=pltpu.CompilerParams(use_tc_tiling_on_sc=False),
    )
    return run(table, indices_flat2d)


def kernel(indices, table):
    idx = indices.astype(jnp.int32).reshape(B // IW, IW)
    out = _lookup(idx, table)
    return out.reshape(indices.shape + (D,))


# speedup vs baseline: 1.4853x; 1.4853x over previous
import jax
import jax.numpy as jnp
from jax import lax
from jax.experimental import pallas as pl
from jax.experimental.pallas import tpu as pltpu
from jax.experimental.pallas import tpu_sc as plsc

V = 1000000
D = 64
B = 4096 * 200

_info = plsc.get_sparse_core_info()
NC = _info.num_cores
NS = _info.num_subcores
NW = NC * NS
B_PER_W = B // NW
CHUNK = 256
CHUNKS = B_PER_W // CHUNK


def _sc_gather(table_hbm, idx_hbm, out_hbm, idx_v, rows_v,
               gsem0, gsem1, ssem0, ssem1):
    wid = lax.axis_index("s") * NC + lax.axis_index("c")
    flat_base = wid * B_PER_W
    gsems = (gsem0, gsem1)
    ssems = (ssem0, ssem1)

    pltpu.sync_copy(idx_hbm.at[pl.ds(flat_base, B_PER_W)], idx_v)

    def fire_rows(g, b):
        base = g * CHUNK

        @pl.loop(0, CHUNK // 16)
        def _grp(q):
            vec = idx_v[pl.ds(base + q * 16, 16)]
            for l in range(16):
                v = vec[l]
                pltpu.async_copy(
                    table_hbm.at[pl.ds(v, 1)],
                    rows_v.at[b].at[pl.ds(q * 16 + l, 1)],
                    gsems[b],
                )

    def drain_rows(b):
        pltpu.make_async_copy(
            table_hbm.at[pl.ds(0, CHUNK)], rows_v.at[b], gsems[b]
        ).wait()

    def store_async(g, b):
        pltpu.async_copy(
            rows_v.at[b],
            out_hbm.at[pl.ds(flat_base + g * CHUNK, CHUNK)],
            ssems[b],
        )

    def drain_store(b):
        pltpu.make_async_copy(
            rows_v.at[b], out_hbm.at[pl.ds(flat_base, CHUNK)], ssems[b]
        ).wait()

    fire_rows(0, 0)

    @pl.loop(0, CHUNKS, step=2)
    def _pair(g):
        drain_rows(0)
        store_async(g, 0)

        @pl.when(g > 0)
        def _():
            drain_store(1)

        fire_rows(g + 1, 1)
        drain_rows(1)
        store_async(g + 1, 1)

        @pl.when(g + 2 < CHUNKS)
        def _():
            drain_store(0)
            fire_rows(g + 2, 0)

    drain_store(0)
    drain_store(1)


@jax.jit
def _lookup(idx_flat, table):
    mesh = plsc.VectorSubcoreMesh(core_axis_name="c", subcore_axis_name="s")
    run = pl.kernel(
        _sc_gather,
        out_type=jax.ShapeDtypeStruct((B, D), jnp.float32),
        mesh=mesh,
        scratch_types=[
            pltpu.VMEM((B_PER_W,), jnp.int32),
            pltpu.VMEM((2, CHUNK, D), jnp.float32),
            pltpu.SemaphoreType.DMA,
            pltpu.SemaphoreType.DMA,
            pltpu.SemaphoreType.DMA,
            pltpu.SemaphoreType.DMA,
        ],
        compiler_params=pltpu.CompilerParams(use_tc_tiling_on_sc=True),
    )
    return run(table, idx_flat)


def kernel(indices, table):
    idx = indices.astype(jnp.int32).reshape(B)
    out = _lookup(idx, table)
    return out.reshape(indices.shape + (D,))
